# Initial kernel scaffold; baseline (speedup 1.0000x reference)
import jax
import jax.numpy as jnp
from jax.experimental import pallas as pl


def kernel(x, edge_index, batch, W1, b1, W2, b2, W3, b3, Wp1, bp1, Wp2, bp2):
    raise NotImplementedError("write your pallas kernel here")


# speedup vs baseline: 34.2304x; 34.2304x over previous
import functools

import jax
import jax.numpy as jnp
from jax import lax
from jax.experimental import pallas as pl
from jax.experimental.pallas import tpu as pltpu
from jax.experimental.pallas import tpu_sc as plsc

N_NODES = 10000
D_IN = 128
HID = 64
NC = 2
NS = 16
NW = NC * NS
CH = 128
BM = 1000
NROWS = 10240
RPT = NROWS // NS

_MESH = plsc.VectorSubcoreMesh(core_axis_name="c", subcore_axis_name="s")
_SC_PARAMS = pltpu.CompilerParams(use_tc_tiling_on_sc=False)


@functools.partial(jax.jit, static_argnums=(2,))
def _sc_deg(dst3, zeros1, nch):
    @functools.partial(
        pl.kernel,
        mesh=_MESH,
        out_type=jax.ShapeDtypeStruct((NC, NROWS), jnp.float32),
        compiler_params=_SC_PARAMS,
        scratch_types=[
            pltpu.VMEM((nch, CH), jnp.int32),
            pltpu.VMEM((CH,), jnp.float32),
            pltpu.VMEM_SHARED((NROWS,), jnp.float32),
            pltpu.SemaphoreType.DMA,
        ],
    )
    def deg_kernel(dst_hbm, zeros_hbm, out_hbm, dstv, ones_v, acc, sd):
        c = lax.axis_index("c")
        s = lax.axis_index("s")
        w = c * NS + s
        pltpu.sync_copy(dst_hbm.at[w], dstv)
        for i in range(CH // 16):
            ones_v[pl.ds(i * 16, 16)] = jnp.ones((16,), jnp.float32)
        pltpu.sync_copy(zeros_hbm.at[pl.ds(s * RPT, RPT)],
                        acc.at[pl.ds(s * RPT, RPT)])
        plsc.subcore_barrier()

        lag = 8

        def fire(j, carry):
            pltpu.async_copy(ones_v, acc.at[dstv.at[j]], sd, add=True)

            @pl.when(j >= lag)
            def _():
                pltpu.make_async_copy(ones_v, acc.at[dstv.at[0]], sd).wait()

            return carry

        lax.fori_loop(0, nch, fire, 0)

        def drain(j, carry):
            pltpu.make_async_copy(ones_v, acc.at[dstv.at[0]], sd).wait()
            return carry

        lax.fori_loop(0, min(lag, nch), drain, 0)
        plsc.subcore_barrier()
        pltpu.sync_copy(acc.at[pl.ds(s * RPT, RPT)],
                        out_hbm.at[c, pl.ds(s * RPT, RPT)])

    return deg_kernel(dst3, zeros1)


@functools.partial(jax.jit, static_argnums=(4,))
def _sc_agg(src3, dst3, y, zeros2, nch):
    @functools.partial(
        pl.kernel,
        mesh=_MESH,
        out_type=jax.ShapeDtypeStruct((NC, NROWS, HID), jnp.float32),
        compiler_params=_SC_PARAMS,
        scratch_types=[
            pltpu.VMEM((nch, CH), jnp.int32),
            pltpu.VMEM((nch, CH), jnp.int32),
            pltpu.VMEM((2, CH, HID), jnp.float32),
            pltpu.VMEM_SHARED((NROWS, HID), jnp.float32),
            pltpu.SemaphoreType.DMA,
            pltpu.SemaphoreType.DMA,
        ],
    )
    def agg_kernel(src_hbm, dst_hbm, y_hbm, zeros_hbm, out_hbm,
                   srcv, dstv, rows, acc, sg0, sg1):
        c = lax.axis_index("c")
        s = lax.axis_index("s")
        w = c * NS + s
        pltpu.sync_copy(src_hbm.at[w], srcv)
        pltpu.sync_copy(dst_hbm.at[w], dstv)
        pltpu.sync_copy(zeros_hbm.at[pl.ds(s * RPT, RPT)],
                        acc.at[pl.ds(s * RPT, RPT)])
        plsc.subcore_barrier()

        pltpu.async_copy(y_hbm.at[srcv.at[0]], rows.at[0], sg0)

        def step(k, carry):
            j0 = 2 * k
            pltpu.async_copy(y_hbm.at[srcv.at[j0 + 1]], rows.at[1], sg1)
            pltpu.make_async_copy(y_hbm.at[pl.ds(0, CH)], rows.at[0], sg0).wait()
            pltpu.sync_copy(rows.at[0], acc.at[dstv.at[j0]], add=True)
            pltpu.async_copy(y_hbm.at[srcv.at[j0 + 2]], rows.at[0], sg0)
            pltpu.make_async_copy(y_hbm.at[pl.ds(0, CH)], rows.at[1], sg1).wait()
            pltpu.sync_copy(rows.at[1], acc.at[dstv.at[j0 + 1]], add=True)
            return carry

        lax.fori_loop(0, (nch - 1) // 2, step, 0)
        pltpu.make_async_copy(y_hbm.at[pl.ds(0, CH)], rows.at[0], sg0).wait()
        pltpu.sync_copy(rows.at[0], acc.at[dstv.at[nch - 1]], add=True)
        plsc.subcore_barrier()
        pltpu.sync_copy(acc.at[pl.ds(s * RPT, RPT)],
                        out_hbm.at[c, pl.ds(s * RPT, RPT)])

    return agg_kernel(src3, dst3, y, zeros2)


def _dinv_block(degt_ref):
    deg = degt_ref[:, 0] + degt_ref[:, 1] + 1.0
    return lax.rsqrt(deg)[:, None]


def _t1_body(x_ref, w_ref, degt_ref, y_ref):
    dinv = _dinv_block(degt_ref)
    xw = jnp.dot(x_ref[...], w_ref[...], preferred_element_type=jnp.float32)
    y_ref[...] = xw * dinv


def _t1(x, W1, degt):
    return pl.pallas_call(
        _t1_body,
        grid=(N_NODES // BM,),
        in_specs=[
            pl.BlockSpec((BM, D_IN), lambda i: (i, 0)),
            pl.BlockSpec((D_IN, HID), lambda i: (0, 0)),
            pl.BlockSpec((BM, 2), lambda i: (i, 0)),
        ],
        out_specs=pl.BlockSpec((BM, HID), lambda i: (i, 0)),
        out_shape=jax.ShapeDtypeStruct((N_NODES, HID), jnp.float32),
    )(x, W1, degt)


def _t23_body(accp_ref, y_ref, b_ref, w_ref, degt_ref, yo_ref):
    dinv = _dinv_block(degt_ref)
    h = jnp.maximum(
        (accp_ref[0] + accp_ref[1] + y_ref[...]) * dinv + b_ref[...], 0.0)
    yo_ref[...] = jnp.dot(
        h, w_ref[...], preferred_element_type=jnp.float32) * dinv


def _t23(accp, y, b_row, W, degt):
    return pl.pallas_call(
        _t23_body,
        grid=(N_NODES // BM,),
        in_specs=[
            pl.BlockSpec((NC, BM, HID), lambda i: (0, i, 0)),
            pl.BlockSpec((BM, HID), lambda i: (i, 0)),
            pl.BlockSpec((1, HID), lambda i: (0, 0)),
            pl.BlockSpec((HID, HID), lambda i: (0, 0)),
            pl.BlockSpec((BM, 2), lambda i: (i, 0)),
        ],
        out_specs=pl.BlockSpec((BM, HID), lambda i: (i, 0)),
        out_shape=jax.ShapeDtypeStruct((N_NODES, HID), jnp.float32),
    )(accp, y, b_row, W, degt)


def _t4_body(accp_ref, y_ref, b_ref, degt_ref, wp1_ref, bp1_ref,
             wp2_ref, bp2_ref, o_ref):
    dinv = _dinv_block(degt_ref)
    h = jnp.maximum(
        (accp_ref[0] + accp_ref[1] + y_ref[...]) * dinv + b_ref[...], 0.0)
    t = jnp.maximum(
        jnp.dot(h, wp1_ref[...], preferred_element_type=jnp.float32)
        + bp1_ref[...], 0.0)
    o_ref[...] = jnp.dot(
        t, wp2_ref[...], preferred_element_type=jnp.float32) + bp2_ref[...]


def _t4(accp, y, b_row, degt, Wp1, bp1_row, Wp2, bp2_row):
    return pl.pallas_call(
        _t4_body,
        grid=(N_NODES // BM,),
        in_specs=[
            pl.BlockSpec((NC, BM, HID), lambda i: (0, i, 0)),
            pl.BlockSpec((BM, HID), lambda i: (i, 0)),
            pl.BlockSpec((1, HID), lambda i: (0, 0)),
            pl.BlockSpec((BM, 2), lambda i: (i, 0)),
            pl.BlockSpec((HID, HID // 2), lambda i: (0, 0)),
            pl.BlockSpec((1, HID // 2), lambda i: (0, 0)),
            pl.BlockSpec((HID // 2, 1), lambda i: (0, 0)),
            pl.BlockSpec((1, 1), lambda i: (0, 0)),
        ],
        out_specs=pl.BlockSpec((BM, 1), lambda i: (i, 0)),
        out_shape=jax.ShapeDtypeStruct((N_NODES, 1), jnp.float32),
    )(accp, y, b_row, degt, Wp1, bp1_row, Wp2, bp2_row)


def kernel(x, edge_index, batch, W1, b1, W2, b2, W3, b3, Wp1, bp1, Wp2, bp2):
    E = edge_index.shape[1]
    ept = -(-E // NW)
    nch = -(-ept // CH)
    if nch % 2 == 0:
        nch += 1
    pad = NW * nch * CH - E
    src = jnp.concatenate(
        [edge_index[0], jnp.zeros((pad,), jnp.int32)])
    dst = jnp.concatenate(
        [edge_index[1], jnp.full((pad,), N_NODES, jnp.int32)])
    src3 = src.reshape(NW, nch, CH)
    dst3 = dst.reshape(NW, nch, CH)
    zeros1 = jnp.zeros((NROWS,), jnp.float32)
    zeros2 = jnp.zeros((NROWS, HID), jnp.float32)

    degp = _sc_deg(dst3, zeros1, nch)
    degt = degp.T

    y1 = _t1(x, W1, degt)
    a1 = _sc_agg(src3, dst3, y1, zeros2, nch)
    y2 = _t23(a1, y1, b1[None], W2, degt)
    a2 = _sc_agg(src3, dst3, y2, zeros2, nch)
    y3 = _t23(a2, y2, b2[None], W3, degt)
    a3 = _sc_agg(src3, dst3, y3, zeros2, nch)
    return _t4(a3, y3, b3[None], degt, Wp1, bp1[None], Wp2, bp2[None])


# speedup vs baseline: 57.5586x; 1.6815x over previous
import functools

import jax
import jax.numpy as jnp
from jax import lax
from jax.experimental import pallas as pl
from jax.experimental.pallas import tpu as pltpu
from jax.experimental.pallas import tpu_sc as plsc

N_NODES = 10000
D_IN = 128
HID = 64
NC = 2
NS = 16
NW = NC * NS
CH = 128
BM = 1000
NROWS = 10240
RPT = NROWS // NS

_MESH = plsc.VectorSubcoreMesh(core_axis_name="c", subcore_axis_name="s")
_SC_PARAMS = pltpu.CompilerParams(use_tc_tiling_on_sc=False)


@functools.partial(jax.jit, static_argnums=(2,))
def _sc_deg(dst3, zeros1, nch):
    @functools.partial(
        pl.kernel,
        mesh=_MESH,
        out_type=jax.ShapeDtypeStruct((NC, NROWS), jnp.float32),
        compiler_params=_SC_PARAMS,
        scratch_types=[
            pltpu.VMEM((nch, CH), jnp.int32),
            pltpu.VMEM((CH,), jnp.float32),
            pltpu.VMEM_SHARED((NROWS,), jnp.float32),
            pltpu.SemaphoreType.DMA,
        ],
    )
    def deg_kernel(dst_hbm, zeros_hbm, out_hbm, dstv, ones_v, acc, sd):
        c = lax.axis_index("c")
        s = lax.axis_index("s")
        w = c * NS + s
        pltpu.sync_copy(dst_hbm.at[w], dstv)
        for i in range(CH // 16):
            ones_v[pl.ds(i * 16, 16)] = jnp.ones((16,), jnp.float32)
        pltpu.sync_copy(zeros_hbm.at[pl.ds(s * RPT, RPT)],
                        acc.at[pl.ds(s * RPT, RPT)])
        plsc.subcore_barrier()

        lag = 8

        def fire(j, carry):
            pltpu.async_copy(ones_v, acc.at[dstv.at[j]], sd, add=True)

            @pl.when(j >= lag)
            def _():
                pltpu.make_async_copy(ones_v, acc.at[dstv.at[0]], sd).wait()

            return carry

        lax.fori_loop(0, nch, fire, 0)

        def drain(j, carry):
            pltpu.make_async_copy(ones_v, acc.at[dstv.at[0]], sd).wait()
            return carry

        lax.fori_loop(0, min(lag, nch), drain, 0)
        plsc.subcore_barrier()
        pltpu.sync_copy(acc.at[pl.ds(s * RPT, RPT)],
                        out_hbm.at[c, pl.ds(s * RPT, RPT)])

    return deg_kernel(dst3, zeros1)


@functools.partial(jax.jit, static_argnums=(4,))
def _sc_agg(src3, dst3, y, zeros2, nch):
    @functools.partial(
        pl.kernel,
        mesh=_MESH,
        out_type=jax.ShapeDtypeStruct((NC, NROWS, HID), jnp.float32),
        compiler_params=_SC_PARAMS,
        scratch_types=[
            pltpu.VMEM((nch, CH), jnp.int32),
            pltpu.VMEM((nch, CH), jnp.int32),
            pltpu.VMEM((4, CH, HID), jnp.float32),
            pltpu.VMEM_SHARED((NROWS, HID), jnp.float32),
            [pltpu.SemaphoreType.DMA] * 4,
            [pltpu.SemaphoreType.DMA] * 4,
        ],
    )
    def agg_kernel(src_hbm, dst_hbm, y_hbm, zeros_hbm, out_hbm,
                   srcv, dstv, rows, acc, gsem, ssem):
        c = lax.axis_index("c")
        s = lax.axis_index("s")
        w = c * NS + s
        pltpu.sync_copy(src_hbm.at[w], srcv)
        pltpu.sync_copy(dst_hbm.at[w], dstv)
        pltpu.sync_copy(zeros_hbm.at[pl.ds(s * RPT, RPT)],
                        acc.at[pl.ds(s * RPT, RPT)])
        plsc.subcore_barrier()

        nk = (nch - 1) // 4

        def wait_g(b):
            pltpu.make_async_copy(y_hbm.at[pl.ds(0, CH)], rows.at[b], gsem[b]).wait()

        def wait_s(b):
            pltpu.make_async_copy(rows.at[b], acc.at[dstv.at[0]], ssem[b]).wait()

        for b in range(3):
            pltpu.async_copy(y_hbm.at[srcv.at[b]], rows.at[b], gsem[b])

        def step(k, carry):
            j0 = 4 * k
            for p in range(4):
                j = j0 + p
                nb = (p + 3) % 4

                @pl.when(j >= 1)
                def _():
                    wait_s(nb)

                @pl.when(j + 3 <= nch - 1)
                def _():
                    pltpu.async_copy(y_hbm.at[srcv.at[j + 3]], rows.at[nb], gsem[nb])

                wait_g(p)
                pltpu.async_copy(rows.at[p], acc.at[dstv.at[j]], ssem[p], add=True)
            return carry

        lax.fori_loop(0, nk, step, 0)
        wait_g(0)
        pltpu.async_copy(rows.at[0], acc.at[dstv.at[nch - 1]], ssem[0], add=True)
        wait_s(3)
        wait_s(0)
        plsc.subcore_barrier()
        pltpu.sync_copy(acc.at[pl.ds(s * RPT, RPT)],
                        out_hbm.at[c, pl.ds(s * RPT, RPT)])

    return agg_kernel(src3, dst3, y, zeros2)


def _dinv_block(degt_ref):
    deg = degt_ref[:, 0] + degt_ref[:, 1] + 1.0
    return lax.rsqrt(deg)[:, None]


def _t1_body(x_ref, w_ref, degt_ref, y_ref):
    dinv = _dinv_block(degt_ref)
    xw = jnp.dot(x_ref[...], w_ref[...], preferred_element_type=jnp.float32)
    y_ref[...] = xw * dinv


def _t1(x, W1, degt):
    return pl.pallas_call(
        _t1_body,
        grid=(N_NODES // BM,),
        in_specs=[
            pl.BlockSpec((BM, D_IN), lambda i: (i, 0)),
            pl.BlockSpec((D_IN, HID), lambda i: (0, 0)),
            pl.BlockSpec((BM, 2), lambda i: (i, 0)),
        ],
        out_specs=pl.BlockSpec((BM, HID), lambda i: (i, 0)),
        out_shape=jax.ShapeDtypeStruct((N_NODES, HID), jnp.float32),
    )(x, W1, degt)


def _t23_body(accp_ref, y_ref, b_ref, w_ref, degt_ref, yo_ref):
    dinv = _dinv_block(degt_ref)
    h = jnp.maximum(
        (accp_ref[0] + accp_ref[1] + y_ref[...]) * dinv + b_ref[...], 0.0)
    yo_ref[...] = jnp.dot(
        h, w_ref[...], preferred_element_type=jnp.float32) * dinv


def _t23(accp, y, b_row, W, degt):
    return pl.pallas_call(
        _t23_body,
        grid=(N_NODES // BM,),
        in_specs=[
            pl.BlockSpec((NC, BM, HID), lambda i: (0, i, 0)),
            pl.BlockSpec((BM, HID), lambda i: (i, 0)),
            pl.BlockSpec((1, HID), lambda i: (0, 0)),
            pl.BlockSpec((HID, HID), lambda i: (0, 0)),
            pl.BlockSpec((BM, 2), lambda i: (i, 0)),
        ],
        out_specs=pl.BlockSpec((BM, HID), lambda i: (i, 0)),
        out_shape=jax.ShapeDtypeStruct((N_NODES, HID), jnp.float32),
    )(accp, y, b_row, W, degt)


def _t4_body(accp_ref, y_ref, b_ref, degt_ref, wp1_ref, bp1_ref,
             wp2_ref, bp2_ref, o_ref):
    dinv = _dinv_block(degt_ref)
    h = jnp.maximum(
        (accp_ref[0] + accp_ref[1] + y_ref[...]) * dinv + b_ref[...], 0.0)
    t = jnp.maximum(
        jnp.dot(h, wp1_ref[...], preferred_element_type=jnp.float32)
        + bp1_ref[...], 0.0)
    o_ref[...] = jnp.dot(
        t, wp2_ref[...], preferred_element_type=jnp.float32) + bp2_ref[...]


def _t4(accp, y, b_row, degt, Wp1, bp1_row, Wp2, bp2_row):
    return pl.pallas_call(
        _t4_body,
        grid=(N_NODES // BM,),
        in_specs=[
            pl.BlockSpec((NC, BM, HID), lambda i: (0, i, 0)),
            pl.BlockSpec((BM, HID), lambda i: (i, 0)),
            pl.BlockSpec((1, HID), lambda i: (0, 0)),
            pl.BlockSpec((BM, 2), lambda i: (i, 0)),
            pl.BlockSpec((HID, HID // 2), lambda i: (0, 0)),
            pl.BlockSpec((1, HID // 2), lambda i: (0, 0)),
            pl.BlockSpec((HID // 2, 1), lambda i: (0, 0)),
            pl.BlockSpec((1, 1), lambda i: (0, 0)),
        ],
        out_specs=pl.BlockSpec((BM, 1), lambda i: (i, 0)),
        out_shape=jax.ShapeDtypeStruct((N_NODES, 1), jnp.float32),
    )(accp, y, b_row, degt, Wp1, bp1_row, Wp2, bp2_row)


def kernel(x, edge_index, batch, W1, b1, W2, b2, W3, b3, Wp1, bp1, Wp2, bp2):
    E = edge_index.shape[1]
    ept = -(-E // NW)
    nch = -(-ept // CH)
    while nch % 4 != 1:
        nch += 1
    pad = NW * nch * CH - E
    pad_iota = jnp.arange(pad, dtype=jnp.int32)
    src = jnp.concatenate([edge_index[0], pad_iota % N_NODES])
    dst = jnp.concatenate(
        [edge_index[1], N_NODES + pad_iota % (NROWS - N_NODES)])
    src3 = src.reshape(NW, nch, CH)
    dst3 = dst.reshape(NW, nch, CH)
    zeros1 = jnp.zeros((NROWS,), jnp.float32)
    zeros2 = jnp.zeros((NROWS, HID), jnp.float32)

    degp = _sc_deg(dst3, zeros1, nch)
    degt = degp.T

    y1 = _t1(x, W1, degt)
    a1 = _sc_agg(src3, dst3, y1, zeros2, nch)
    y2 = _t23(a1, y1, b1[None], W2, degt)
    a2 = _sc_agg(src3, dst3, y2, zeros2, nch)
    y3 = _t23(a2, y2, b2[None], W3, degt)
    a3 = _sc_agg(src3, dst3, y3, zeros2, nch)
    return _t4(a3, y3, b3[None], degt, Wp1, bp1[None], Wp2, bp2[None])


# speedup vs baseline: 57.9984x; 1.0076x over previous
import functools

import jax
import jax.numpy as jnp
from jax import lax
from jax.experimental import pallas as pl
from jax.experimental.pallas import tpu as pltpu
from jax.experimental.pallas import tpu_sc as plsc

N_NODES = 10000
D_IN = 128
HID = 64
NC = 2
NS = 16
NW = NC * NS
CH = 128
BM = 1000
NROWS = 10240
RPT = NROWS // NS

_MESH = plsc.VectorSubcoreMesh(core_axis_name="c", subcore_axis_name="s")
_SC_PARAMS = pltpu.CompilerParams(use_tc_tiling_on_sc=False)


@functools.partial(jax.jit, static_argnums=(2,))
def _sc_deg(edges3, zeros1, nch):
    @functools.partial(
        pl.kernel,
        mesh=_MESH,
        out_type=jax.ShapeDtypeStruct((NC, NROWS), jnp.float32),
        compiler_params=_SC_PARAMS,
        scratch_types=[
            pltpu.VMEM((nch, CH), jnp.int32),
            pltpu.VMEM((CH,), jnp.float32),
            pltpu.VMEM_SHARED((NROWS,), jnp.float32),
            pltpu.SemaphoreType.DMA,
            pltpu.SemaphoreType.DMA,
        ],
    )
    def deg_kernel(edges_hbm, zeros_hbm, out_hbm, dstv, ones_v, acc, sd, zsem):
        c = lax.axis_index("c")
        s = lax.axis_index("s")
        w = c * NS + s
        pltpu.async_copy(zeros_hbm.at[pl.ds(s * RPT, RPT)],
                         acc.at[pl.ds(s * RPT, RPT)], zsem)
        pltpu.sync_copy(edges_hbm.at[w, 1], dstv)
        for i in range(CH // 16):
            ones_v[pl.ds(i * 16, 16)] = jnp.ones((16,), jnp.float32)
        pltpu.make_async_copy(zeros_hbm.at[pl.ds(0, RPT)],
                              acc.at[pl.ds(0, RPT)], zsem).wait()
        plsc.subcore_barrier()

        lag = 8

        def fire(j, carry):
            pltpu.async_copy(ones_v, acc.at[dstv.at[j]], sd, add=True)

            @pl.when(j >= lag)
            def _():
                pltpu.make_async_copy(ones_v, acc.at[dstv.at[0]], sd).wait()

            return carry

        lax.fori_loop(0, nch, fire, 0)

        def drain(j, carry):
            pltpu.make_async_copy(ones_v, acc.at[dstv.at[0]], sd).wait()
            return carry

        lax.fori_loop(0, min(lag, nch), drain, 0)
        plsc.subcore_barrier()
        pltpu.sync_copy(acc.at[pl.ds(s * RPT, RPT)],
                        out_hbm.at[c, pl.ds(s * RPT, RPT)])

    return deg_kernel(edges3, zeros1)


@functools.partial(jax.jit, static_argnums=(3,))
def _sc_agg(edges3, y, zeros2, nch):
    @functools.partial(
        pl.kernel,
        mesh=_MESH,
        out_type=jax.ShapeDtypeStruct((NC, NROWS, HID), jnp.float32),
        compiler_params=_SC_PARAMS,
        scratch_types=[
            pltpu.VMEM((2, nch, CH), jnp.int32),
            pltpu.VMEM((4, CH, HID), jnp.float32),
            pltpu.VMEM_SHARED((NROWS, HID), jnp.float32),
            [pltpu.SemaphoreType.DMA] * 4,
            [pltpu.SemaphoreType.DMA] * 4,
            pltpu.SemaphoreType.DMA,
        ],
    )
    def agg_kernel(edges_hbm, y_hbm, zeros_hbm, out_hbm,
                   idxv, rows, acc, gsem, ssem, zsem):
        c = lax.axis_index("c")
        s = lax.axis_index("s")
        w = c * NS + s
        srcv = idxv.at[0]
        dstv = idxv.at[1]
        pltpu.async_copy(zeros_hbm.at[pl.ds(s * RPT, RPT)],
                         acc.at[pl.ds(s * RPT, RPT)], zsem)
        pltpu.sync_copy(edges_hbm.at[w], idxv)

        nk = (nch - 1) // 4

        def wait_g(b):
            pltpu.make_async_copy(y_hbm.at[pl.ds(0, CH)], rows.at[b], gsem[b]).wait()

        def wait_s(b):
            pltpu.make_async_copy(rows.at[b], acc.at[dstv.at[0]], ssem[b]).wait()

        for b in range(3):
            pltpu.async_copy(y_hbm.at[srcv.at[b]], rows.at[b], gsem[b])
        pltpu.make_async_copy(zeros_hbm.at[pl.ds(0, RPT)],
                              acc.at[pl.ds(0, RPT)], zsem).wait()
        plsc.subcore_barrier()

        def step(k, carry):
            j0 = 4 * k
            for p in range(4):
                j = j0 + p
                nb = (p + 3) % 4

                @pl.when(j >= 1)
                def _():
                    wait_s(nb)

                @pl.when(j + 3 <= nch - 1)
                def _():
                    pltpu.async_copy(y_hbm.at[srcv.at[j + 3]], rows.at[nb], gsem[nb])

                wait_g(p)
                pltpu.async_copy(rows.at[p], acc.at[dstv.at[j]], ssem[p], add=True)
            return carry

        lax.fori_loop(0, nk, step, 0)
        wait_g(0)
        pltpu.async_copy(rows.at[0], acc.at[dstv.at[nch - 1]], ssem[0], add=True)
        wait_s(3)
        wait_s(0)
        plsc.subcore_barrier()
        pltpu.sync_copy(acc.at[pl.ds(s * RPT, RPT)],
                        out_hbm.at[c, pl.ds(s * RPT, RPT)])

    return agg_kernel(edges3, y, zeros2)


def _dinv_block(degt_ref):
    deg = degt_ref[:, 0] + degt_ref[:, 1] + 1.0
    return lax.rsqrt(deg)[:, None]


def _t1_body(x_ref, w_ref, degt_ref, y_ref):
    dinv = _dinv_block(degt_ref)
    xw = jnp.dot(x_ref[...], w_ref[...], preferred_element_type=jnp.float32)
    y_ref[...] = xw * dinv


def _t1(x, W1, degt):
    return pl.pallas_call(
        _t1_body,
        grid=(N_NODES // BM,),
        in_specs=[
            pl.BlockSpec((BM, D_IN), lambda i: (i, 0)),
            pl.BlockSpec((D_IN, HID), lambda i: (0, 0)),
            pl.BlockSpec((BM, 2), lambda i: (i, 0)),
        ],
        out_specs=pl.BlockSpec((BM, HID), lambda i: (i, 0)),
        out_shape=jax.ShapeDtypeStruct((N_NODES, HID), jnp.float32),
    )(x, W1, degt)


def _t23_body(accp_ref, y_ref, b_ref, w_ref, degt_ref, yo_ref):
    dinv = _dinv_block(degt_ref)
    h = jnp.maximum(
        (accp_ref[0] + accp_ref[1] + y_ref[...]) * dinv + b_ref[...], 0.0)
    yo_ref[...] = jnp.dot(
        h, w_ref[...], preferred_element_type=jnp.float32) * dinv


def _t23(accp, y, b_row, W, degt):
    return pl.pallas_call(
        _t23_body,
        grid=(N_NODES // BM,),
        in_specs=[
            pl.BlockSpec((NC, BM, HID), lambda i: (0, i, 0)),
            pl.BlockSpec((BM, HID), lambda i: (i, 0)),
            pl.BlockSpec((1, HID), lambda i: (0, 0)),
            pl.BlockSpec((HID, HID), lambda i: (0, 0)),
            pl.BlockSpec((BM, 2), lambda i: (i, 0)),
        ],
        out_specs=pl.BlockSpec((BM, HID), lambda i: (i, 0)),
        out_shape=jax.ShapeDtypeStruct((N_NODES, HID), jnp.float32),
    )(accp, y, b_row, W, degt)


def _t4_body(accp_ref, y_ref, b_ref, degt_ref, wp1_ref, bp1_ref,
             wp2_ref, bp2_ref, o_ref):
    dinv = _dinv_block(degt_ref)
    h = jnp.maximum(
        (accp_ref[0] + accp_ref[1] + y_ref[...]) * dinv + b_ref[...], 0.0)
    t = jnp.maximum(
        jnp.dot(h, wp1_ref[...], preferred_element_type=jnp.float32)
        + bp1_ref[...], 0.0)
    o_ref[...] = jnp.dot(
        t, wp2_ref[...], preferred_element_type=jnp.float32) + bp2_ref[...]


def _t4(accp, y, b_row, degt, Wp1, bp1_row, Wp2, bp2_row):
    return pl.pallas_call(
        _t4_body,
        grid=(N_NODES // BM,),
        in_specs=[
            pl.BlockSpec((NC, BM, HID), lambda i: (0, i, 0)),
            pl.BlockSpec((BM, HID), lambda i: (i, 0)),
            pl.BlockSpec((1, HID), lambda i: (0, 0)),
            pl.BlockSpec((BM, 2), lambda i: (i, 0)),
            pl.BlockSpec((HID, HID // 2), lambda i: (0, 0)),
            pl.BlockSpec((1, HID // 2), lambda i: (0, 0)),
            pl.BlockSpec((HID // 2, 1), lambda i: (0, 0)),
            pl.BlockSpec((1, 1), lambda i: (0, 0)),
        ],
        out_specs=pl.BlockSpec((BM, 1), lambda i: (i, 0)),
        out_shape=jax.ShapeDtypeStruct((N_NODES, 1), jnp.float32),
    )(accp, y, b_row, degt, Wp1, bp1_row, Wp2, bp2_row)


def kernel(x, edge_index, batch, W1, b1, W2, b2, W3, b3, Wp1, bp1, Wp2, bp2):
    E = edge_index.shape[1]
    ept = -(-E // NW)
    nch = -(-ept // CH)
    while nch % 4 != 1:
        nch += 1
    pad = NW * nch * CH - E
    pad_iota = jnp.arange(pad, dtype=jnp.int32)
    src = jnp.concatenate([edge_index[0], pad_iota % N_NODES])
    dst = jnp.concatenate(
        [edge_index[1], N_NODES + pad_iota % (NROWS - N_NODES)])
    edges3 = jnp.stack(
        [src.reshape(NW, nch, CH), dst.reshape(NW, nch, CH)], axis=1)
    zeros1 = jnp.zeros((NROWS,), jnp.float32)
    zeros2 = jnp.zeros((NROWS, HID), jnp.float32)

    degp = _sc_deg(edges3, zeros1, nch)
    degt = degp.T

    y1 = _t1(x, W1, degt)
    a1 = _sc_agg(edges3, y1, zeros2, nch)
    y2 = _t23(a1, y1, b1[None], W2, degt)
    a2 = _sc_agg(edges3, y2, zeros2, nch)
    y3 = _t23(a2, y2, b2[None], W3, degt)
    a3 = _sc_agg(edges3, y3, zeros2, nch)
    return _t4(a3, y3, b3[None], degt, Wp1, bp1[None], Wp2, bp2[None])


# speedup vs baseline: 59.1801x; 1.0204x over previous
import functools

import jax
import jax.numpy as jnp
from jax import lax
from jax.experimental import pallas as pl
from jax.experimental.pallas import tpu as pltpu
from jax.experimental.pallas import tpu_sc as plsc

N_NODES = 10000
D_IN = 128
HID = 64
NC = 2
NS = 16
NW = NC * NS
CH = 128
BM = 2000
NROWS = 10240
RPT = NROWS // NS

_MESH = plsc.VectorSubcoreMesh(core_axis_name="c", subcore_axis_name="s")
_SC_PARAMS = pltpu.CompilerParams(use_tc_tiling_on_sc=False)


@functools.partial(jax.jit, static_argnums=(2,))
def _sc_deg(edges3, zeros1, nch):
    @functools.partial(
        pl.kernel,
        mesh=_MESH,
        out_type=jax.ShapeDtypeStruct((NC, NROWS), jnp.float32),
        compiler_params=_SC_PARAMS,
        scratch_types=[
            pltpu.VMEM((nch, CH), jnp.int32),
            pltpu.VMEM((CH,), jnp.float32),
            pltpu.VMEM_SHARED((NROWS,), jnp.float32),
            pltpu.SemaphoreType.DMA,
            pltpu.SemaphoreType.DMA,
        ],
    )
    def deg_kernel(edges_hbm, zeros_hbm, out_hbm, dstv, ones_v, acc, sd, zsem):
        c = lax.axis_index("c")
        s = lax.axis_index("s")
        w = c * NS + s
        pltpu.async_copy(zeros_hbm.at[pl.ds(s * RPT, RPT)],
                         acc.at[pl.ds(s * RPT, RPT)], zsem)
        pltpu.sync_copy(edges_hbm.at[w, 1], dstv)
        for i in range(CH // 16):
            ones_v[pl.ds(i * 16, 16)] = jnp.ones((16,), jnp.float32)
        pltpu.make_async_copy(zeros_hbm.at[pl.ds(0, RPT)],
                              acc.at[pl.ds(0, RPT)], zsem).wait()
        plsc.subcore_barrier()

        lag = 8

        def fire(j, carry):
            pltpu.async_copy(ones_v, acc.at[dstv.at[j]], sd, add=True)

            @pl.when(j >= lag)
            def _():
                pltpu.make_async_copy(ones_v, acc.at[dstv.at[0]], sd).wait()

            return carry

        lax.fori_loop(0, nch, fire, 0)

        def drain(j, carry):
            pltpu.make_async_copy(ones_v, acc.at[dstv.at[0]], sd).wait()
            return carry

        lax.fori_loop(0, min(lag, nch), drain, 0)
        plsc.subcore_barrier()
        pltpu.sync_copy(acc.at[pl.ds(s * RPT, RPT)],
                        out_hbm.at[c, pl.ds(s * RPT, RPT)])

    return deg_kernel(edges3, zeros1)


@functools.partial(jax.jit, static_argnums=(3,))
def _sc_agg(edges3, y, zeros2, nch):
    @functools.partial(
        pl.kernel,
        mesh=_MESH,
        out_type=jax.ShapeDtypeStruct((NC, NROWS, HID), jnp.float32),
        compiler_params=_SC_PARAMS,
        scratch_types=[
            pltpu.VMEM((2, nch, CH), jnp.int32),
            pltpu.VMEM((4, CH, HID), jnp.float32),
            pltpu.VMEM_SHARED((NROWS, HID), jnp.float32),
            [pltpu.SemaphoreType.DMA] * 4,
            [pltpu.SemaphoreType.DMA] * 4,
            pltpu.SemaphoreType.DMA,
        ],
    )
    def agg_kernel(edges_hbm, y_hbm, zeros_hbm, out_hbm,
                   idxv, rows, acc, gsem, ssem, zsem):
        c = lax.axis_index("c")
        s = lax.axis_index("s")
        w = c * NS + s
        srcv = idxv.at[0]
        dstv = idxv.at[1]
        pltpu.async_copy(zeros_hbm.at[pl.ds(s * RPT, RPT)],
                         acc.at[pl.ds(s * RPT, RPT)], zsem)
        pltpu.sync_copy(edges_hbm.at[w], idxv)

        nk = (nch - 1) // 4

        def wait_g(b):
            pltpu.make_async_copy(y_hbm.at[pl.ds(0, CH)], rows.at[b], gsem[b]).wait()

        def wait_s(b):
            pltpu.make_async_copy(rows.at[b], acc.at[dstv.at[0]], ssem[b]).wait()

        for b in range(3):
            pltpu.async_copy(y_hbm.at[srcv.at[b]], rows.at[b], gsem[b])
        pltpu.make_async_copy(zeros_hbm.at[pl.ds(0, RPT)],
                              acc.at[pl.ds(0, RPT)], zsem).wait()
        plsc.subcore_barrier()

        def step(k, carry):
            j0 = 4 * k
            for p in range(4):
                j = j0 + p
                nb = (p + 3) % 4

                @pl.when(j >= 1)
                def _():
                    wait_s(nb)

                @pl.when(j + 3 <= nch - 1)
                def _():
                    pltpu.async_copy(y_hbm.at[srcv.at[j + 3]], rows.at[nb], gsem[nb])

                wait_g(p)
                pltpu.async_copy(rows.at[p], acc.at[dstv.at[j]], ssem[p], add=True)
            return carry

        lax.fori_loop(0, nk, step, 0)
        wait_g(0)
        pltpu.async_copy(rows.at[0], acc.at[dstv.at[nch - 1]], ssem[0], add=True)
        wait_s(3)
        wait_s(0)
        plsc.subcore_barrier()
        pltpu.sync_copy(acc.at[pl.ds(s * RPT, RPT)],
                        out_hbm.at[c, pl.ds(s * RPT, RPT)])

    return agg_kernel(edges3, y, zeros2)


def _dinv_block(degt_ref):
    deg = degt_ref[:, 0] + degt_ref[:, 1] + 1.0
    return lax.rsqrt(deg)[:, None]


def _t1a_body(x_ref, w_ref, xw_ref):
    xw_ref[...] = jnp.dot(
        x_ref[...], w_ref[...], preferred_element_type=jnp.float32)


def _t1a(x, W1):
    return pl.pallas_call(
        _t1a_body,
        grid=(N_NODES // BM,),
        in_specs=[
            pl.BlockSpec((BM, D_IN), lambda i: (i, 0)),
            pl.BlockSpec((D_IN, HID), lambda i: (0, 0)),
        ],
        out_specs=pl.BlockSpec((BM, HID), lambda i: (i, 0)),
        out_shape=jax.ShapeDtypeStruct((N_NODES, HID), jnp.float32),
    )(x, W1)


def _t1b_body(xw_ref, degt_ref, y_ref):
    y_ref[...] = xw_ref[...] * _dinv_block(degt_ref)


def _t1b(xw, degt):
    return pl.pallas_call(
        _t1b_body,
        grid=(N_NODES // BM,),
        in_specs=[
            pl.BlockSpec((BM, HID), lambda i: (i, 0)),
            pl.BlockSpec((BM, 2), lambda i: (i, 0)),
        ],
        out_specs=pl.BlockSpec((BM, HID), lambda i: (i, 0)),
        out_shape=jax.ShapeDtypeStruct((N_NODES, HID), jnp.float32),
    )(xw, degt)


def _t23_body(accp_ref, y_ref, b_ref, w_ref, degt_ref, yo_ref):
    dinv = _dinv_block(degt_ref)
    h = jnp.maximum(
        (accp_ref[0] + accp_ref[1] + y_ref[...]) * dinv + b_ref[...], 0.0)
    yo_ref[...] = jnp.dot(
        h, w_ref[...], preferred_element_type=jnp.float32) * dinv


def _t23(accp, y, b_row, W, degt):
    return pl.pallas_call(
        _t23_body,
        grid=(N_NODES // BM,),
        in_specs=[
            pl.BlockSpec((NC, BM, HID), lambda i: (0, i, 0)),
            pl.BlockSpec((BM, HID), lambda i: (i, 0)),
            pl.BlockSpec((1, HID), lambda i: (0, 0)),
            pl.BlockSpec((HID, HID), lambda i: (0, 0)),
            pl.BlockSpec((BM, 2), lambda i: (i, 0)),
        ],
        out_specs=pl.BlockSpec((BM, HID), lambda i: (i, 0)),
        out_shape=jax.ShapeDtypeStruct((N_NODES, HID), jnp.float32),
    )(accp, y, b_row, W, degt)


def _t4_body(accp_ref, y_ref, b_ref, degt_ref, wp1_ref, bp1_ref,
             wp2_ref, bp2_ref, o_ref):
    dinv = _dinv_block(degt_ref)
    h = jnp.maximum(
        (accp_ref[0] + accp_ref[1] + y_ref[...]) * dinv + b_ref[...], 0.0)
    t = jnp.maximum(
        jnp.dot(h, wp1_ref[...], preferred_element_type=jnp.float32)
        + bp1_ref[...], 0.0)
    o_ref[...] = jnp.dot(
        t, wp2_ref[...], preferred_element_type=jnp.float32) + bp2_ref[...]


def _t4(accp, y, b_row, degt, Wp1, bp1_row, Wp2, bp2_row):
    return pl.pallas_call(
        _t4_body,
        grid=(N_NODES // BM,),
        in_specs=[
            pl.BlockSpec((NC, BM, HID), lambda i: (0, i, 0)),
            pl.BlockSpec((BM, HID), lambda i: (i, 0)),
            pl.BlockSpec((1, HID), lambda i: (0, 0)),
            pl.BlockSpec((BM, 2), lambda i: (i, 0)),
            pl.BlockSpec((HID, HID // 2), lambda i: (0, 0)),
            pl.BlockSpec((1, HID // 2), lambda i: (0, 0)),
            pl.BlockSpec((HID // 2, 1), lambda i: (0, 0)),
            pl.BlockSpec((1, 1), lambda i: (0, 0)),
        ],
        out_specs=pl.BlockSpec((BM, 1), lambda i: (i, 0)),
        out_shape=jax.ShapeDtypeStruct((N_NODES, 1), jnp.float32),
    )(accp, y, b_row, degt, Wp1, bp1_row, Wp2, bp2_row)


def kernel(x, edge_index, batch, W1, b1, W2, b2, W3, b3, Wp1, bp1, Wp2, bp2):
    E = edge_index.shape[1]
    ept = -(-E // NW)
    nch = -(-ept // CH)
    while nch % 4 != 1:
        nch += 1
    pad = NW * nch * CH - E
    pad_iota = jnp.arange(pad, dtype=jnp.int32)
    src = jnp.concatenate([edge_index[0], pad_iota % N_NODES])
    dst = jnp.concatenate(
        [edge_index[1], N_NODES + pad_iota % (NROWS - N_NODES)])
    edges3 = jnp.stack(
        [src.reshape(NW, nch, CH), dst.reshape(NW, nch, CH)], axis=1)
    zeros1 = jnp.zeros((NROWS,), jnp.float32)
    zeros2 = jnp.zeros((NROWS, HID), jnp.float32)

    xw1 = _t1a(x, W1)
    degp = _sc_deg(edges3, zeros1, nch)
    degt = degp.T

    y1 = _t1b(xw1, degt)
    a1 = _sc_agg(edges3, y1, zeros2, nch)
    y2 = _t23(a1, y1, b1[None], W2, degt)
    a2 = _sc_agg(edges3, y2, zeros2, nch)
    y3 = _t23(a2, y2, b2[None], W3, degt)
    a3 = _sc_agg(edges3, y3, zeros2, nch)
    return _t4(a3, y3, b3[None], degt, Wp1, bp1[None], Wp2, bp2[None])
